# baseline (speedup 1.0000x reference)
import functools
import math

import jax
import jax.numpy as jnp
from jax import lax
from jax.experimental import pallas as pl
from jax.experimental.pallas import tpu as pltpu
from jax.experimental.pallas import tpu_sc as plsc

_NUM_BUCKETS = 32
_NUM_HEADS = 16
_N = 2048
_GR = 16
_L = 16


def _bucket_of_n(n: int) -> int:
    if n < 16:
        return n
    return min(31, 16 + int(math.floor(16.0 * math.log2(n / 16.0))))


_BUCKETS = [_bucket_of_n(n) for n in range(32)]


def kernel(query_len, key_len, relative_attention_bias):
    del query_len, key_len
    table_flat = relative_attention_bias.reshape(-1)
    mesh = plsc.VectorSubcoreMesh(core_axis_name="c", subcore_axis_name="s")

    @functools.partial(
        pl.kernel,
        mesh=mesh,
        out_type=jax.ShapeDtypeStruct((_NUM_HEADS, _N, _N), jnp.float32),
        scratch_types=[
            pltpu.VMEM((_NUM_BUCKETS * _NUM_HEADS,), jnp.float32),
            pltpu.VMEM((_GR * 64,), jnp.float32),
            pltpu.VMEM((_GR * _N,), jnp.float32),
            pltpu.VMEM((_GR * _N,), jnp.float32),
            pltpu.SemaphoreType.DMA,
            pltpu.SemaphoreType.DMA,
            pltpu.SemaphoreType.DMA,
        ],
    )
    def _sc_kernel(table_hbm, out_hbm, tab_v, pat_v, stg_a, stg_b,
                   sem_t, sem_a, sem_b):
        wid = lax.axis_index("s") * 2 + lax.axis_index("c")
        head = wid // 2
        half = wid % 2
        q_base = half * (_N // 2)

        pltpu.async_copy(table_hbm, tab_v, sem_t).wait()

        iota = lax.iota(jnp.int32, _L)
        hvec = iota * 0 + head

        def dvec(n):
            row = tab_v[pl.ds(_BUCKETS[n] * _NUM_HEADS, _L)]
            return row.at[hvec].get(mode="promise_in_bounds")

        c0v = dvec(0)
        c31v = dvec(31)

        for dq in range(_GR):
            for cw in range(4):
                n = jnp.clip(dq - (iota + cw * _L) + 32, 0, 31)
                acc = c0v
                for b in range(1, 32):
                    acc = jnp.where(n == b, dvec(b), acc)
                pat_v[pl.ds(dq * 64 + cw * _L, _L)] = acc

        def init_row(i, _):
            for cw in range(_N // _L):
                val = jnp.where(cw * _L < q_base, c31v, c0v)
                stg_a[pl.ds(i * _N + cw * _L, _L)] = val
                stg_b[pl.ds(i * _N + cw * _L, _L)] = val
            return 0

        lax.fori_loop(0, _GR, init_row, 0)

        def paint(stg, g, erase):
            col0 = q_base + g * _GR - 32

            def touch_row(i, _):
                if erase:
                    for cw in range(4):
                        col = col0 - 32 + cw * _L

                        @pl.when(jnp.logical_and(col >= 0, col <= _N - _L))
                        def _():
                            stg[pl.ds(i * _N + col, _L)] = c31v
                for cw in range(4):
                    col = col0 + cw * _L
                    src = pat_v[pl.ds(i * 64 + cw * _L, _L)]

                    @pl.when(jnp.logical_and(col >= 0, col <= _N - _L))
                    def _():
                        stg[pl.ds(i * _N + col, _L)] = src
                return 0

            lax.fori_loop(0, _GR, touch_row, 0)

        paint(stg_a, 0, False)
        paint(stg_b, 1, False)

        n_groups = (_N // 2) // _GR

        def send(stg, q0, sem):
            handles = [
                pltpu.async_copy(
                    stg.at[pl.ds(r * _N, _N)],
                    out_hbm.at[head, q0 + r, :],
                    sem,
                )
                for r in range(_GR)
            ]
            for h in handles:
                h.wait()

        def grp(g, carry):
            del carry
            q0 = q_base + g * _GR

            @pl.when(g % 2 == 0)
            def _():
                send(stg_a, q0, sem_a)

            @pl.when(g % 2 == 1)
            def _():
                send(stg_b, q0, sem_b)

            @pl.when(g + 2 < n_groups)
            def _():
                @pl.when(g % 2 == 0)
                def _():
                    paint(stg_a, g + 2, True)

                @pl.when(g % 2 == 1)
                def _():
                    paint(stg_b, g + 2, True)

            return 0

        lax.fori_loop(0, n_groups, grp, 0)

    return _sc_kernel(table_flat)


# speedup vs baseline: 1.4874x; 1.4874x over previous
import math

import jax
import jax.numpy as jnp
from jax.experimental import pallas as pl
from jax.experimental.pallas import tpu as pltpu

_NUM_BUCKETS = 32
_NUM_HEADS = 16
_Q = 2048
_K = 2048
_BQ = 1024
_SB = 128


def _bucket_of_n(n: int) -> int:
    if n < 16:
        return n
    return min(31, 16 + int(math.floor(16.0 * math.log2(n / 16.0))))


_BUCKETS = [_bucket_of_n(n) for n in range(32)]


def _bias_kernel(table_ref, out_ref, band_ref):
    h = pl.program_id(0)
    qi = pl.program_id(1)

    @pl.when(qi == 0)
    def _():
        dq = jax.lax.broadcasted_iota(jnp.int32, (_SB, 2 * _SB), 0)
        dk = jax.lax.broadcasted_iota(jnp.int32, (_SB, 2 * _SB), 1)
        n = jnp.clip(dq - dk + _SB, 0, 31)
        acc = jnp.full((_SB, 2 * _SB), table_ref[0, 0, 0], jnp.float32)
        for b in range(1, 32):
            acc = jnp.where(n == b, table_ref[0, 0, _BUCKETS[b]], acc)
        band_ref[...] = acc

    c0 = table_ref[0, 0, 0]
    c31 = table_ref[0, 0, 31]

    col = jax.lax.broadcasted_iota(jnp.int32, (_SB, _K), 1)
    n_sub = _BQ // _SB
    for r in range(n_sub):
        qb = qi * n_sub + r
        rows = slice(r * _SB, (r + 1) * _SB)
        out_ref[0, rows, :] = jnp.where(col >= qb * _SB, c0, c31)
        out_ref[0, rows, pl.ds(qb * _SB, _SB)] = band_ref[:, _SB:]
        if r > 0:
            out_ref[0, rows, pl.ds((qb - 1) * _SB, _SB)] = band_ref[:, :_SB]
        else:
            @pl.when(qi > 0)
            def _():
                out_ref[0, rows, pl.ds((qb - 1) * _SB, _SB)] = band_ref[:, :_SB]


def kernel(query_len, key_len, relative_attention_bias):
    del query_len, key_len
    table = jnp.transpose(relative_attention_bias).reshape(_NUM_HEADS, 1, _NUM_BUCKETS)
    grid = (_NUM_HEADS, _Q // _BQ)
    return pl.pallas_call(
        _bias_kernel,
        grid=grid,
        in_specs=[
            pl.BlockSpec((1, 1, _NUM_BUCKETS), lambda h, qi: (h, 0, 0)),
        ],
        out_specs=pl.BlockSpec((1, _BQ, _K), lambda h, qi: (h, qi, 0)),
        out_shape=jax.ShapeDtypeStruct((_NUM_HEADS, _Q, _K), jnp.float32),
        scratch_shapes=[pltpu.VMEM((_SB, 2 * _SB), jnp.float32)],
    )(table)
